# baseline (speedup 1.0000x reference)
import functools

import jax
import jax.numpy as jnp
from jax import lax
from jax.experimental import pallas as pl
from jax.experimental.pallas import tpu as pltpu
from jax.experimental.pallas import tpu_sc as plsc

LANES = 16
G = 16


@functools.lru_cache(maxsize=None)
def _build_sc_pool(num_batch, seq_len, vocab, hidden):
    info = plsc.get_sparse_core_info()
    nc, ns = info.num_cores, info.num_subcores
    nw = nc * ns
    tokens = num_batch * seq_len
    assert tokens % nw == 0
    tok_per_w = tokens // nw
    assert tok_per_w % G == 0
    ng = tok_per_w // G
    seg = nw // num_batch
    chunks = hidden // LANES

    mesh = plsc.VectorSubcoreMesh(core_axis_name="c", subcore_axis_name="s")

    def body(ids_hbm, table_hbm, out_hbm, idx_v, rows_v, acc_v, sem):
        wid = lax.axis_index("s") * nc + lax.axis_index("c")
        b = wid % num_batch
        s = wid // num_batch
        base = b * seq_len + s * tok_per_w
        pltpu.sync_copy(ids_hbm.at[pl.ds(base, tok_per_w)], idx_v)

        def _zero(c, carry):
            acc_v[pl.ds(c * LANES, LANES)] = jnp.zeros((LANES,), jnp.float32)
            return carry

        lax.fori_loop(0, chunks, _zero, 0)

        for g in range(ng):
            iv = idx_v[pl.ds(g * G, G)]
            pltpu.async_copy(table_hbm.at[iv], rows_v, sem).wait()

            def _acc(c, carry):
                off = c * LANES
                a = acc_v[pl.ds(off, LANES)]
                for r in range(G):
                    a = a + rows_v[r, pl.ds(off, LANES)]
                acc_v[pl.ds(off, LANES)] = a
                return carry

            lax.fori_loop(0, chunks, _acc, 0)

        pltpu.sync_copy(acc_v, out_hbm.at[s, b])

    return pl.kernel(
        body,
        mesh=mesh,
        out_type=jax.ShapeDtypeStruct((seg, num_batch, hidden), jnp.float32),
        scratch_types=[
            pltpu.VMEM((tok_per_w,), jnp.int32),
            pltpu.VMEM((G, hidden), jnp.float32),
            pltpu.VMEM((hidden,), jnp.float32),
            pltpu.SemaphoreType.DMA,
        ],
    )


def _finish_body(seg, seq_len, p_ref, a_ref, b_ref, o_ref):
    pooled = p_ref[0]
    for i in range(1, seg):
        pooled = pooled + p_ref[i]
    pooled = pooled * (1.0 / seq_len)
    t = jnp.dot(pooled, a_ref[...], preferred_element_type=jnp.float32)
    o_ref[...] = jnp.dot(t, b_ref[...], preferred_element_type=jnp.float32)


@jax.jit
def kernel(input_ids, embedding, lora_A, lora_B):
    num_batch, seq_len = input_ids.shape
    vocab, hidden = embedding.shape
    ids = input_ids.reshape(-1).astype(jnp.int32)
    sc_pool = _build_sc_pool(num_batch, seq_len, vocab, hidden)
    partials = sc_pool(ids, embedding)
    seg = partials.shape[0]
    return pl.pallas_call(
        functools.partial(_finish_body, seg, seq_len),
        out_shape=jax.ShapeDtypeStruct((num_batch, hidden), jnp.float32),
    )(partials, lora_A, lora_B)


# speedup vs baseline: 1.4025x; 1.4025x over previous
import functools

import jax
import jax.numpy as jnp
from jax import lax
from jax.experimental import pallas as pl
from jax.experimental.pallas import tpu as pltpu
from jax.experimental.pallas import tpu_sc as plsc

LANES = 16
G = 8
UNROLL = 2


@functools.lru_cache(maxsize=None)
def _build_sc_pool(num_batch, seq_len, vocab, hidden):
    info = plsc.get_sparse_core_info()
    nc, ns = info.num_cores, info.num_subcores
    nw = nc * ns
    tokens = num_batch * seq_len
    assert tokens % nw == 0
    tok_per_w = tokens // nw
    assert tok_per_w % G == 0
    ng = tok_per_w // G
    seg = nw // num_batch
    chunks = hidden // LANES
    assert chunks % UNROLL == 0

    mesh = plsc.VectorSubcoreMesh(core_axis_name="c", subcore_axis_name="s")

    def body(ids_hbm, table_hbm, out_hbm, idx2, rows0, rows1, acc_v, sem0, sem1):
        wid = lax.axis_index("s") * nc + lax.axis_index("c")
        b = wid % num_batch
        s = wid // num_batch
        base = b * seq_len + s * tok_per_w
        row0 = pl.multiple_of(base // G, 8)
        pltpu.sync_copy(ids_hbm.at[pl.ds(row0, ng)], idx2)

        def _zero(c, carry):
            off = c * (LANES * UNROLL)
            for u in range(UNROLL):
                acc_v[pl.ds(off + u * LANES, LANES)] = jnp.zeros(
                    (LANES,), jnp.float32)
            return carry

        lax.fori_loop(0, chunks // UNROLL, _zero, 0)

        bufs = (rows0, rows1)
        sems = (sem0, sem1)
        handles = [None] * ng
        handles[0] = pltpu.async_copy(table_hbm.at[idx2.at[0]], rows0, sem0)
        if ng > 1:
            handles[1] = pltpu.async_copy(table_hbm.at[idx2.at[1]], rows1, sem1)

        for g in range(ng):
            handles[g].wait()
            buf = bufs[g % 2]

            def _acc(c, carry, buf=buf):
                off = c * (LANES * UNROLL)
                for u in range(UNROLL):
                    o = off + u * LANES
                    t0 = buf[0, pl.ds(o, LANES)] + buf[1, pl.ds(o, LANES)]
                    t1 = buf[2, pl.ds(o, LANES)] + buf[3, pl.ds(o, LANES)]
                    t2 = buf[4, pl.ds(o, LANES)] + buf[5, pl.ds(o, LANES)]
                    t3 = buf[6, pl.ds(o, LANES)] + buf[7, pl.ds(o, LANES)]
                    plsc.addupdate(acc_v.at[pl.ds(o, LANES)],
                                   (t0 + t1) + (t2 + t3))
                return carry

            lax.fori_loop(0, chunks // UNROLL, _acc, 0)

            if g + 2 < ng:
                handles[g + 2] = pltpu.async_copy(
                    table_hbm.at[idx2.at[g + 2]], bufs[g % 2], sems[g % 2])

        pltpu.sync_copy(acc_v, out_hbm.at[s, b])

    return pl.kernel(
        body,
        mesh=mesh,
        out_type=jax.ShapeDtypeStruct((seg, num_batch, hidden), jnp.float32),
        scratch_types=[
            pltpu.VMEM((ng, G), jnp.int32),
            pltpu.VMEM((G, hidden), jnp.float32),
            pltpu.VMEM((G, hidden), jnp.float32),
            pltpu.VMEM((hidden,), jnp.float32),
            pltpu.SemaphoreType.DMA,
            pltpu.SemaphoreType.DMA,
        ],
    )


def _finish_body(seg, seq_len, p_ref, a_ref, b_ref, o_ref):
    pooled = p_ref[0]
    for i in range(1, seg):
        pooled = pooled + p_ref[i]
    pooled = pooled * (1.0 / seq_len)
    t = jnp.dot(pooled, a_ref[...], preferred_element_type=jnp.float32)
    o_ref[...] = jnp.dot(t, b_ref[...], preferred_element_type=jnp.float32)


@jax.jit
def kernel(input_ids, embedding, lora_A, lora_B):
    num_batch, seq_len = input_ids.shape
    vocab, hidden = embedding.shape
    ids = input_ids.reshape(-1).astype(jnp.int32).reshape(-1, G)
    sc_pool = _build_sc_pool(num_batch, seq_len, vocab, hidden)
    partials = sc_pool(ids, embedding)
    seg = partials.shape[0]
    return pl.pallas_call(
        functools.partial(_finish_body, seg, seq_len),
        out_shape=jax.ShapeDtypeStruct((num_batch, hidden), jnp.float32),
    )(partials, lora_A, lora_B)


# speedup vs baseline: 1.9145x; 1.3650x over previous
import functools

import jax
import jax.numpy as jnp
from jax import lax
from jax.experimental import pallas as pl
from jax.experimental.pallas import tpu as pltpu
from jax.experimental.pallas import tpu_sc as plsc

LANES = 16
G = 8
UNROLL = 4


@functools.lru_cache(maxsize=None)
def _build_sc_pool(num_batch, seq_len, vocab, hidden):
    info = plsc.get_sparse_core_info()
    nc, ns = info.num_cores, info.num_subcores
    nw = nc * ns
    tokens = num_batch * seq_len
    assert tokens % nw == 0
    tok_per_w = tokens // nw
    assert tok_per_w % G == 0
    ng = tok_per_w // G
    seg = nw // num_batch
    chunks = hidden // LANES
    assert chunks % UNROLL == 0

    mesh = plsc.VectorSubcoreMesh(core_axis_name="c", subcore_axis_name="s")

    def body(ids_hbm, table_hbm, out_hbm, idx2, rows0, rows1, acc_v, sem0, sem1):
        wid = lax.axis_index("s") * nc + lax.axis_index("c")
        b = wid % num_batch
        s = wid // num_batch
        base = b * seq_len + s * tok_per_w
        row0 = pl.multiple_of(base // G, 8)
        pltpu.sync_copy(ids_hbm.at[pl.ds(row0, ng)], idx2)

        @plsc.parallel_loop(0, chunks, unroll=UNROLL)
        def _zero(c):
            acc_v[pl.ds(c * LANES, LANES)] = jnp.zeros((LANES,), jnp.float32)

        bufs = (rows0, rows1)
        sems = (sem0, sem1)
        handles = [None] * ng
        handles[0] = pltpu.async_copy(table_hbm.at[idx2.at[0]], rows0, sem0)
        if ng > 1:
            handles[1] = pltpu.async_copy(table_hbm.at[idx2.at[1]], rows1, sem1)

        for g in range(ng):
            handles[g].wait()
            buf = bufs[g % 2]

            @plsc.parallel_loop(0, chunks, unroll=UNROLL)
            def _acc(c, buf=buf):
                o = c * LANES
                t0 = buf[0, pl.ds(o, LANES)] + buf[1, pl.ds(o, LANES)]
                t1 = buf[2, pl.ds(o, LANES)] + buf[3, pl.ds(o, LANES)]
                t2 = buf[4, pl.ds(o, LANES)] + buf[5, pl.ds(o, LANES)]
                t3 = buf[6, pl.ds(o, LANES)] + buf[7, pl.ds(o, LANES)]
                plsc.addupdate(acc_v.at[pl.ds(o, LANES)],
                               (t0 + t1) + (t2 + t3))

            if g + 2 < ng:
                handles[g + 2] = pltpu.async_copy(
                    table_hbm.at[idx2.at[g + 2]], bufs[g % 2], sems[g % 2])

        pltpu.sync_copy(acc_v, out_hbm.at[s, b])

    return pl.kernel(
        body,
        mesh=mesh,
        out_type=jax.ShapeDtypeStruct((seg, num_batch, hidden), jnp.float32),
        scratch_types=[
            pltpu.VMEM((ng, G), jnp.int32),
            pltpu.VMEM((G, hidden), jnp.float32),
            pltpu.VMEM((G, hidden), jnp.float32),
            pltpu.VMEM((hidden,), jnp.float32),
            pltpu.SemaphoreType.DMA,
            pltpu.SemaphoreType.DMA,
        ],
    )


def _finish_body(seg, seq_len, p_ref, a_ref, b_ref, o_ref):
    pooled = p_ref[0]
    for i in range(1, seg):
        pooled = pooled + p_ref[i]
    pooled = pooled * (1.0 / seq_len)
    t = jnp.dot(pooled, a_ref[...], preferred_element_type=jnp.float32)
    o_ref[...] = jnp.dot(t, b_ref[...], preferred_element_type=jnp.float32)


@jax.jit
def kernel(input_ids, embedding, lora_A, lora_B):
    num_batch, seq_len = input_ids.shape
    vocab, hidden = embedding.shape
    ids = input_ids.reshape(-1).astype(jnp.int32).reshape(-1, G)
    sc_pool = _build_sc_pool(num_batch, seq_len, vocab, hidden)
    partials = sc_pool(ids, embedding)
    seg = partials.shape[0]
    return pl.pallas_call(
        functools.partial(_finish_body, seg, seq_len),
        out_shape=jax.ShapeDtypeStruct((num_batch, hidden), jnp.float32),
    )(partials, lora_A, lora_B)
